# baseline (speedup 1.0000x reference)
import functools

import jax
import jax.numpy as jnp
from jax import lax
from jax.experimental import pallas as pl
from jax.experimental.pallas import tpu as pltpu
from jax.experimental.pallas import tpu_sc as plsc

B, L, D = 4096, 201, 128
V = 3

NC, NS = 2, 16
NW = NC * NS
BPW = B // NW
RPW = BPW * L
CH = 128
NCHUNK = RPW // CH


def _body(ids_hbm, len_hbm, tab_hbm, out_hbm, clen_hbm,
          ids_v, len_v, len1_v, eofidx_v, eofsrc_v, eofbuf_v, buf_v, sem):
    wid = lax.axis_index("s") * NC + lax.axis_index("c")
    base_b = wid * BPW
    base_r = wid * RPW

    pltpu.sync_copy(ids_hbm.at[pl.ds(base_r, RPW)], ids_v)
    pltpu.sync_copy(len_hbm.at[pl.ds(base_b, BPW)], len_v)

    eof_row = jnp.full((16,), V, dtype=jnp.int32)
    for k in range(BPW // 16):
        lv = len_v[pl.ds(k * 16, 16)]
        jb = lax.iota(jnp.int32, 16) + (k * 16)
        eofidx_v[pl.ds(k * 16, 16)] = base_r + jb * L + lv
        eofsrc_v[pl.ds(k * 16, 16)] = eof_row
        len1_v[pl.ds(k * 16, 16)] = lv + 1
    pltpu.sync_copy(len1_v, clen_hbm.at[pl.ds(base_b, BPW)])

    pltpu.async_copy(tab_hbm.at[eofsrc_v], eofbuf_v, sem).wait()

    def chunk(c, carry):
        idx = ids_v.at[pl.ds(c * CH, CH)]
        pltpu.async_copy(tab_hbm.at[idx], buf_v, sem).wait()
        pltpu.sync_copy(buf_v, out_hbm.at[pl.ds(base_r + c * CH, CH)])
        return carry

    lax.fori_loop(0, NCHUNK, chunk, 0)

    pltpu.async_copy(eofbuf_v, out_hbm.at[eofidx_v], sem).wait()


_sc_call = pl.kernel(
    _body,
    out_type=(
        jax.ShapeDtypeStruct((B * L, D), jnp.float32),
        jax.ShapeDtypeStruct((B,), jnp.int32),
    ),
    mesh=plsc.VectorSubcoreMesh(core_axis_name="c", subcore_axis_name="s"),
    scratch_types=[
        pltpu.VMEM((RPW,), jnp.int32),
        pltpu.VMEM((BPW,), jnp.int32),
        pltpu.VMEM((BPW,), jnp.int32),
        pltpu.VMEM((BPW,), jnp.int32),
        pltpu.VMEM((BPW,), jnp.int32),
        pltpu.VMEM((BPW, D), jnp.float32),
        pltpu.VMEM((CH, D), jnp.float32),
        pltpu.SemaphoreType.DMA,
    ],
)


@jax.jit
def kernel(word_ids, lengths, table, eof_embedding):
    ids_flat = word_ids.reshape(B * L)
    table4 = jnp.concatenate([table, eof_embedding], axis=0)
    rep_flat, char_len = _sc_call(ids_flat, lengths, table4)
    return rep_flat.reshape(B, L, D), char_len


# speedup vs baseline: 13.2291x; 13.2291x over previous
import functools

import jax
import jax.numpy as jnp
from jax import lax
from jax.experimental import pallas as pl
from jax.experimental.pallas import tpu as pltpu
from jax.experimental.pallas import tpu_sc as plsc

B, L, D = 4096, 201, 128
V = 3

NC, NS = 2, 16
NW = NC * NS
BPW = B // NW
RPW = BPW * L
CH = 128
NCHUNK = RPW // CH


def _body(ids_hbm, len_hbm, tab_hbm, out_hbm, clen_hbm,
          ids_v, len_v, len1_v, eofidx_v, eofsrc_v, eofbuf_v, buf_v, tab_sh, sem):
    sid = lax.axis_index("s")
    wid = sid * NC + lax.axis_index("c")
    base_b = wid * BPW
    base_r = wid * RPW

    @pl.when(sid == 0)
    def _():
        pltpu.sync_copy(tab_hbm, tab_sh)

    pltpu.sync_copy(ids_hbm.at[pl.ds(base_r, RPW)], ids_v)
    pltpu.sync_copy(len_hbm.at[pl.ds(base_b, BPW)], len_v)
    plsc.subcore_barrier()

    eof_row = jnp.full((16,), V, dtype=jnp.int32)
    for k in range(BPW // 16):
        lv = len_v[pl.ds(k * 16, 16)]
        jb = lax.iota(jnp.int32, 16) + (k * 16)
        eofidx_v[pl.ds(k * 16, 16)] = base_r + jb * L + lv
        eofsrc_v[pl.ds(k * 16, 16)] = eof_row
        len1_v[pl.ds(k * 16, 16)] = lv + 1
    pltpu.sync_copy(len1_v, clen_hbm.at[pl.ds(base_b, BPW)])

    pltpu.async_copy(tab_sh.at[eofsrc_v], eofbuf_v, sem).wait()

    def chunk(c, carry):
        idx = ids_v.at[pl.ds(c * CH, CH)]
        pltpu.async_copy(tab_sh.at[idx], buf_v, sem).wait()
        pltpu.sync_copy(buf_v, out_hbm.at[pl.ds(base_r + c * CH, CH)])
        return carry

    lax.fori_loop(0, NCHUNK, chunk, 0)

    pltpu.async_copy(eofbuf_v, out_hbm.at[eofidx_v], sem).wait()


_sc_call = pl.kernel(
    _body,
    out_type=(
        jax.ShapeDtypeStruct((B * L, D), jnp.float32),
        jax.ShapeDtypeStruct((B,), jnp.int32),
    ),
    mesh=plsc.VectorSubcoreMesh(core_axis_name="c", subcore_axis_name="s"),
    scratch_types=[
        pltpu.VMEM((RPW,), jnp.int32),
        pltpu.VMEM((BPW,), jnp.int32),
        pltpu.VMEM((BPW,), jnp.int32),
        pltpu.VMEM((BPW,), jnp.int32),
        pltpu.VMEM((BPW,), jnp.int32),
        pltpu.VMEM((BPW, D), jnp.float32),
        pltpu.VMEM((CH, D), jnp.float32),
        pltpu.VMEM_SHARED((V + 1, D), jnp.float32),
        pltpu.SemaphoreType.DMA,
    ],
)


@jax.jit
def kernel(word_ids, lengths, table, eof_embedding):
    ids_flat = word_ids.reshape(B * L)
    table4 = jnp.concatenate([table, eof_embedding], axis=0)
    rep_flat, char_len = _sc_call(ids_flat, lengths, table4)
    return rep_flat.reshape(B, L, D), char_len


# speedup vs baseline: 14.7094x; 1.1119x over previous
import functools

import jax
import jax.numpy as jnp
from jax import lax
from jax.experimental import pallas as pl
from jax.experimental.pallas import tpu as pltpu
from jax.experimental.pallas import tpu_sc as plsc

B, L, D = 4096, 201, 128
V = 3

NC, NS = 2, 16
NW = NC * NS
BPW = B // NW
RPW = BPW * L
CH = 96
NCHUNK = RPW // CH
NBUF = 4
NR = NCHUNK // NBUF


def _body(ids_hbm, len_hbm, tab_hbm, out_hbm, clen_hbm,
          ids_v, len_v, len1_v, eofidx_v, eofsrc_v, eofbuf_v,
          b0, b1, b2, b3, tab_sh,
          g0, g1, g2, g3, w0, w1, w2, w3, esem):
    bufs = (b0, b1, b2, b3)
    gsem = (g0, g1, g2, g3)
    wsem = (w0, w1, w2, w3)

    sid = lax.axis_index("s")
    wid = sid * NC + lax.axis_index("c")
    base_b = wid * BPW
    base_r = wid * RPW

    @pl.when(sid == 0)
    def _():
        pltpu.sync_copy(tab_hbm, tab_sh)

    pltpu.sync_copy(ids_hbm.at[pl.ds(base_r, RPW)], ids_v)
    pltpu.sync_copy(len_hbm.at[pl.ds(base_b, BPW)], len_v)
    plsc.subcore_barrier()

    eof_row = jnp.full((16,), V, dtype=jnp.int32)
    for k in range(BPW // 16):
        lv = len_v[pl.ds(k * 16, 16)]
        jb = lax.iota(jnp.int32, 16) + (k * 16)
        eofidx_v[pl.ds(k * 16, 16)] = base_r + jb * L + lv
        eofsrc_v[pl.ds(k * 16, 16)] = eof_row
        len1_v[pl.ds(k * 16, 16)] = lv + 1
    pltpu.sync_copy(len1_v, clen_hbm.at[pl.ds(base_b, BPW)])

    pltpu.async_copy(tab_sh.at[eofsrc_v], eofbuf_v, esem).wait()

    def start_gather(cc, b):
        idx = ids_v.at[pl.ds(cc * CH, CH)]
        pltpu.async_copy(tab_sh.at[idx], bufs[b], gsem[b])

    def wait_gather(b):
        pltpu.make_async_copy(
            tab_sh.at[ids_v.at[pl.ds(0, CH)]], bufs[b], gsem[b]).wait()

    def start_write(cc, b):
        pltpu.async_copy(bufs[b], out_hbm.at[pl.ds(base_r + cc * CH, CH)],
                         wsem[b])

    def wait_write(b):
        pltpu.make_async_copy(
            bufs[b], out_hbm.at[pl.ds(0, CH)], wsem[b]).wait()

    for b in range(NBUF):
        start_gather(b, b)

    def round_body(r, carry):
        for b in range(NBUF):
            wait_gather(b)
            start_write(r * NBUF + b, b)

        @pl.when(r < NR - 1)
        def _():
            for b in range(NBUF):
                wait_write(b)
                start_gather((r + 1) * NBUF + b, b)

        return carry

    lax.fori_loop(0, NR, round_body, 0)
    for b in range(NBUF):
        wait_write(b)

    pltpu.async_copy(eofbuf_v, out_hbm.at[eofidx_v], esem).wait()


_sc_call = pl.kernel(
    _body,
    out_type=(
        jax.ShapeDtypeStruct((B * L, D), jnp.float32),
        jax.ShapeDtypeStruct((B,), jnp.int32),
    ),
    mesh=plsc.VectorSubcoreMesh(core_axis_name="c", subcore_axis_name="s"),
    scratch_types=[
        pltpu.VMEM((RPW,), jnp.int32),
        pltpu.VMEM((BPW,), jnp.int32),
        pltpu.VMEM((BPW,), jnp.int32),
        pltpu.VMEM((BPW,), jnp.int32),
        pltpu.VMEM((BPW,), jnp.int32),
        pltpu.VMEM((BPW, D), jnp.float32),
        pltpu.VMEM((CH, D), jnp.float32),
        pltpu.VMEM((CH, D), jnp.float32),
        pltpu.VMEM((CH, D), jnp.float32),
        pltpu.VMEM((CH, D), jnp.float32),
        pltpu.VMEM_SHARED((V + 1, D), jnp.float32),
        pltpu.SemaphoreType.DMA,
        pltpu.SemaphoreType.DMA,
        pltpu.SemaphoreType.DMA,
        pltpu.SemaphoreType.DMA,
        pltpu.SemaphoreType.DMA,
        pltpu.SemaphoreType.DMA,
        pltpu.SemaphoreType.DMA,
        pltpu.SemaphoreType.DMA,
        pltpu.SemaphoreType.DMA,
    ],
)


@jax.jit
def kernel(word_ids, lengths, table, eof_embedding):
    ids_flat = word_ids.reshape(B * L)
    table4 = jnp.concatenate([table, eof_embedding], axis=0)
    rep_flat, char_len = _sc_call(ids_flat, lengths, table4)
    return rep_flat.reshape(B, L, D), char_len
